# baseline (speedup 1.0000x reference)
import functools

import jax
import jax.numpy as jnp
from jax import lax
from jax.experimental import pallas as pl
from jax.experimental.pallas import tpu as pltpu
from jax.experimental.pallas import tpu_sc as plsc

NC = 2
NS = 16
NW = NC * NS
D = 64
CH = 512


@functools.partial(jax.jit, static_argnames=("n_chunks",))
def _gather_rows(idx3, table, n_chunks):
    B = NW * n_chunks * CH
    mesh = plsc.VectorSubcoreMesh(core_axis_name="c", subcore_axis_name="s")

    @functools.partial(
        pl.kernel,
        out_type=jax.ShapeDtypeStruct((B, D), jnp.float32),
        mesh=mesh,
        scratch_types=[
            pltpu.VMEM((n_chunks * CH,), jnp.int32),
            pltpu.VMEM((2, CH, D), jnp.float32),
            pltpu.SemaphoreType.DMA,
        ],
        compiler_params=pltpu.CompilerParams(use_tc_tiling_on_sc=False),
    )
    def k(idx_hbm, table_hbm, out_hbm, idx_v, rows_v, gsem):
        wid = lax.axis_index("s") * NC + lax.axis_index("c")
        base = wid * (n_chunks * CH)
        pltpu.sync_copy(idx_hbm.at[wid], idx_v)

        pltpu.async_copy(table_hbm.at[idx_v.at[pl.ds(0, CH)]], rows_v.at[0], gsem)

        def body(j, _):
            buf = lax.rem(j, 2)
            nxt = lax.rem(j + 1, 2)

            @pl.when(j + 1 < n_chunks)
            def _():
                pltpu.async_copy(
                    table_hbm.at[idx_v.at[pl.ds((j + 1) * CH, CH)]],
                    rows_v.at[nxt],
                    gsem,
                )

            pltpu.make_async_copy(
                table_hbm.at[idx_v.at[pl.ds(j * CH, CH)]], rows_v.at[buf], gsem
            ).wait()
            pltpu.sync_copy(rows_v.at[buf], out_hbm.at[pl.ds(base + j * CH, CH)])
            return ()

        lax.fori_loop(0, n_chunks, body, (), unroll=False)

    return k(idx3, table)


def kernel(ids, embeddings):
    batch, hist = ids.shape
    B = batch * hist
    n_chunks = B // (NW * CH)
    idx3 = ids.reshape(NW, n_chunks * CH)
    out = _gather_rows(idx3, embeddings, n_chunks)
    return out.reshape(batch, hist, D)
